# Initial kernel scaffold; baseline (speedup 1.0000x reference)
import jax
import jax.numpy as jnp
from jax.experimental import pallas as pl


def kernel(arg, weight):
    raise NotImplementedError("write your pallas kernel here")


# speedup vs baseline: 5.8382x; 5.8382x over previous
import functools

import jax
import jax.numpy as jnp
from jax import lax
from jax.experimental import pallas as pl
from jax.experimental.pallas import tpu as pltpu
from jax.experimental.pallas import tpu_sc as plsc

NUM_ROWS = 16384
NUM_COLS = 200
EMB = 3
N_IDX = NUM_ROWS * NUM_COLS
N_OUT = N_IDX * EMB

NC = 2
NS = 16
NW = NC * NS
LANES = 16

PER_W = N_IDX // NW
CHUNK = 10_240
NCH = PER_W // CHUNK
NITER = CHUNK // LANES


def _sc_body(arg_hbm, w_hbm, out_hbm,
             idx0, idx1, ob0, ob1, wtab,
             sem_i0, sem_i1, sem_o0, sem_o1):
    wid = lax.axis_index("s") * NC + lax.axis_index("c")
    ibase = wid * PER_W
    obase = wid * (PER_W * EMB)

    pltpu.sync_copy(w_hbm, wtab)

    idx_bufs = (idx0, idx1)
    out_bufs = (ob0, ob1)
    sems_i = (sem_i0, sem_i1)
    sems_o = (sem_o0, sem_o1)

    p3 = jnp.arange(LANES, dtype=jnp.int32) * 3

    def start_idx(c):
        return pltpu.async_copy(
            arg_hbm.at[pl.ds(ibase + c * CHUNK, CHUNK)],
            idx_bufs[c % 2], sems_i[c % 2])

    def compute(idx_ref, out_ref):
        @plsc.parallel_loop(0, NITER, 1, unroll=8)
        def _(i):
            av = idx_ref[pl.ds(i * LANES, LANES)]
            b3 = av * 3
            s0 = i * (LANES * 3)
            for k in range(3):
                g = plsc.load_gather(wtab, [b3 + k])
                plsc.store_scatter(out_ref, [s0 + p3 + k], g)

    h_out = [None, None]
    h_idx = start_idx(0)
    for c in range(NCH):
        h_next = start_idx(c + 1) if c + 1 < NCH else None
        h_idx.wait()
        if h_out[c % 2] is not None:
            h_out[c % 2].wait()
        compute(idx_bufs[c % 2], out_bufs[c % 2])
        h_out[c % 2] = pltpu.async_copy(
            out_bufs[c % 2],
            out_hbm.at[pl.ds(obase + c * (CHUNK * EMB), CHUNK * EMB)],
            sems_o[c % 2])
        h_idx = h_next
    for h in h_out:
        if h is not None:
            h.wait()


@functools.partial(jax.jit, static_argnames=())
def _sc_lookup(arg_flat, w_pad):
    mesh = plsc.VectorSubcoreMesh(core_axis_name="c", subcore_axis_name="s")
    f = pl.kernel(
        _sc_body,
        out_type=jax.ShapeDtypeStruct((N_OUT,), jnp.float32),
        mesh=mesh,
        scratch_types=[
            pltpu.VMEM((CHUNK,), jnp.int32),
            pltpu.VMEM((CHUNK,), jnp.int32),
            pltpu.VMEM((CHUNK * EMB,), jnp.float32),
            pltpu.VMEM((CHUNK * EMB,), jnp.float32),
            pltpu.VMEM((32,), jnp.float32),
            pltpu.SemaphoreType.DMA,
            pltpu.SemaphoreType.DMA,
            pltpu.SemaphoreType.DMA,
            pltpu.SemaphoreType.DMA,
        ],
        compiler_params=pltpu.CompilerParams(needs_layout_passes=False),
    )
    return f(arg_flat, w_pad)


def kernel(arg, weight):
    arg_flat = arg.reshape(-1).astype(jnp.int32)
    w_pad = jnp.pad(weight.reshape(-1), (0, 2))
    out = _sc_lookup(arg_flat, w_pad)
    return out.reshape(NUM_ROWS, NUM_COLS, EMB)


# speedup vs baseline: 431.9145x; 73.9804x over previous
import jax
import jax.numpy as jnp
from jax.experimental import pallas as pl
from jax.experimental.pallas import tpu as pltpu

NUM_ROWS = 16384
NUM_COLS = 200
EMB = 3
NVOC = 10

BI = 1024
GRID = NUM_ROWS // BI


def _tc_body(w_ref, a_ref, o_ref):
    a = a_ref[...]
    p0 = jnp.full(a.shape, w_ref[0, 0], jnp.float32)
    p1 = jnp.full(a.shape, w_ref[1, 0], jnp.float32)
    p2 = jnp.full(a.shape, w_ref[2, 0], jnp.float32)
    for r in range(1, NVOC):
        m = a == r
        p0 = jnp.where(m, w_ref[0, r], p0)
        p1 = jnp.where(m, w_ref[1, r], p1)
        p2 = jnp.where(m, w_ref[2, r], p2)
    o_ref[0] = p0
    o_ref[1] = p1
    o_ref[2] = p2


def kernel(arg, weight):
    a_t = arg.T.astype(jnp.int32)
    w_t = weight.T
    out_t = pl.pallas_call(
        _tc_body,
        grid=(GRID,),
        in_specs=[
            pl.BlockSpec(memory_space=pltpu.SMEM),
            pl.BlockSpec((NUM_COLS, BI), lambda i: (0, i)),
        ],
        out_specs=pl.BlockSpec((EMB, NUM_COLS, BI), lambda i: (0, 0, i)),
        out_shape=jax.ShapeDtypeStruct((EMB, NUM_COLS, NUM_ROWS), jnp.float32),
        compiler_params=pltpu.CompilerParams(
            dimension_semantics=("arbitrary",),
        ),
    )(w_t, a_t)
    return out_t.transpose(2, 1, 0)
